# baseline (speedup 1.0000x reference)
import functools

import jax
import jax.numpy as jnp
from jax import lax
from jax.experimental import pallas as pl
from jax.experimental.pallas import tpu as pltpu
from jax.experimental.pallas import tpu_sc as plsc

NUM_EMB = 32000
EMB_DIM = 1024
M_BASE = 20011
V_TEXT = 32000
S = 4096
G = 16
H = EMB_DIM // G

_MODS = [M_BASE + 2 * i + 1 for i in range(G)]
_OFFS = [0]
for _m in _MODS:
    _OFFS.append(_OFFS[-1] + _m)
_OE_ROWS = _OFFS[-1]

NW = 32
TS = S // NW
SUB = 32
NSUB = TS // SUB
ROWS_PER_IDXROW = 8


def _sc_body(tok_hbm, ww_hbm, oe_hbm, x_out, wd_out,
             tok_v, idx_v, widx_v, oe_rows, w_rows, sem0, sem1):
    wid = lax.axis_index("s") * 2 + lax.axis_index("c")
    base = wid * TS

    @pl.when(wid == 0)
    def _():
        tok_v[pl.ds(0, 16)] = jnp.zeros((16,), jnp.int32)
        pltpu.sync_copy(tok_hbm.at[pl.ds(0, TS)], tok_v.at[pl.ds(8, TS)])

    @pl.when(wid != 0)
    def _():
        pltpu.sync_copy(tok_hbm.at[pl.ds(base - 8, TS + 8)],
                        tok_v.at[pl.ds(0, TS + 8)])

    lanes = lax.iota(jnp.int32, 16)

    def _hash_step(j, _):
        off0 = 8 + j * 16
        t0 = plsc.load_gather(tok_v, [lanes + off0])
        t1 = plsc.load_gather(tok_v, [lanes + (off0 - 1)])
        t2 = plsc.load_gather(tok_v, [lanes + (off0 - 2)])
        for g in range(G):
            mod = _MODS[g]
            w1 = pow(V_TEXT, 1, mod)
            acc = t0 % mod
            acc = (acc + (t1 * w1) % mod) % mod
            if g >= 8:
                w2 = pow(V_TEXT, 2, mod)
                acc = (acc + (t2 * w2) % mod) % mod
            ids = acc + _OFFS[g]
            p = lanes * 16 + (j * 256 + g)
            plsc.store_scatter(idx_v, [p >> 7, p & 127], ids)
        return 0

    lax.fori_loop(0, TS // 16, _hash_step, 0)

    for sub in range(NSUB):
        for h2 in range(SUB // 16):
            v = plsc.load_gather(tok_v, [lanes + (8 + sub * SUB + h2 * 16)])
            widx_v[sub, pl.ds(h2 * 16, 16)] = v

    nrow = SUB * G // 128
    for sub in range(NSUB):
        r0 = sub * nrow
        copies = [
            pltpu.async_copy(oe_hbm.at[idx_v.at[r0 + r]],
                             oe_rows.at[pl.ds(r * 128, 128)], sem0)
            for r in range(nrow)
        ]
        cw = pltpu.async_copy(ww_hbm.at[widx_v.at[sub]], w_rows, sem1)
        for c in copies:
            c.wait()
        cw.wait()
        pltpu.sync_copy(oe_rows,
                        x_out.at[pl.ds((base + sub * SUB) * G, SUB * G)])
        pltpu.sync_copy(w_rows, wd_out.at[pl.ds(base + sub * SUB, SUB)])


_sc_gather = functools.partial(
    pl.kernel,
    out_type=(jax.ShapeDtypeStruct((S * G, H), jnp.float32),
              jax.ShapeDtypeStruct((S, EMB_DIM), jnp.float32)),
    mesh=plsc.VectorSubcoreMesh(core_axis_name="c", subcore_axis_name="s"),
    scratch_types=[
        pltpu.VMEM((TS + 16,), jnp.int32),
        pltpu.VMEM((TS * G // 128, 128), jnp.int32),
        pltpu.VMEM((NSUB, SUB), jnp.int32),
        pltpu.VMEM((SUB * G, H), jnp.float32),
        pltpu.VMEM((SUB, EMB_DIM), jnp.float32),
        pltpu.SemaphoreType.DMA,
        pltpu.SemaphoreType.DMA,
    ],
    compiler_params=pltpu.CompilerParams(needs_layout_passes=False,
                                         use_tc_tiling_on_sc=False),
)(_sc_body)


BS = 256


def _tc_body(tok_ref, x_ref, w_ref, wd_ref, o_ref):
    t = tok_ref[...]
    m = ((t != 0) & (t != 1) & (t != 2)).astype(jnp.float32)
    y = jnp.dot(x_ref[...], w_ref[...], preferred_element_type=jnp.float32)
    o_ref[...] = (wd_ref[...] + y * m) * (1.0 / 17.0)


def _tc_combine(tokens, x, wp, wd):
    return pl.pallas_call(
        _tc_body,
        grid=(S // BS,),
        in_specs=[
            pl.BlockSpec((BS, 1), lambda i: (i, 0)),
            pl.BlockSpec((BS, G * H), lambda i: (i, 0)),
            pl.BlockSpec((G * H, EMB_DIM), lambda i: (0, 0)),
            pl.BlockSpec((BS, EMB_DIM), lambda i: (i, 0)),
        ],
        out_specs=pl.BlockSpec((BS, EMB_DIM), lambda i: (i, 0)),
        out_shape=jax.ShapeDtypeStruct((S, EMB_DIM), jnp.float32),
    )(tokens.reshape(S, 1), x, wp, wd)


def kernel(input_ids, word_weight, oe_weight, oe_projection):
    tokens = input_ids.astype(jnp.int32)
    x, wd = _sc_gather(tokens, word_weight, oe_weight)
    wp = oe_projection.reshape(G * H, EMB_DIM)
    return _tc_combine(tokens, x.reshape(S, G * H), wp, wd)


# speedup vs baseline: 1.5147x; 1.5147x over previous
import functools

import jax
import jax.numpy as jnp
from jax import lax
from jax.experimental import pallas as pl
from jax.experimental.pallas import tpu as pltpu
from jax.experimental.pallas import tpu_sc as plsc

NUM_EMB = 32000
EMB_DIM = 1024
M_BASE = 20011
V_TEXT = 32000
S = 4096
G = 16
H = EMB_DIM // G

_MODS = [M_BASE + 2 * i + 1 for i in range(G)]
_OFFS = [0]
for _m in _MODS:
    _OFFS.append(_OFFS[-1] + _m)
_OE_ROWS = _OFFS[-1]

NW = 32
TS = S // NW

_MESH = plsc.VectorSubcoreMesh(core_axis_name="c", subcore_axis_name="s")


WSUB = 32
WNSUB = TS // WSUB
WNBUF = 2


def _sc_word_body(tok_hbm, ww_hbm, wd_out, tok_v, widx_v, w_rows, sems):
    wid = lax.axis_index("s") * 2 + lax.axis_index("c")
    base = wid * TS
    pltpu.sync_copy(tok_hbm.at[pl.ds(base, TS)], tok_v)

    lanes = lax.iota(jnp.int32, 16)
    for sub in range(WNSUB):
        for h2 in range(WSUB // 16):
            v = plsc.load_gather(tok_v, [lanes + (sub * WSUB + h2 * 16)])
            widx_v[sub, pl.ds(h2 * 16, 16)] = v

    def _issue(sub, b):
        return pltpu.async_copy(ww_hbm.at[widx_v.at[sub]], w_rows.at[b],
                                sems.at[b])

    copies = {0: _issue(0, 0)}
    if WNSUB > 1:
        copies[1] = _issue(1, 1)
    for sub in range(WNSUB):
        b = sub % WNBUF
        copies[sub].wait()
        pltpu.sync_copy(w_rows.at[b], wd_out.at[pl.ds(base + sub * WSUB, WSUB)])
        if sub + WNBUF < WNSUB:
            copies[sub + WNBUF] = _issue(sub + WNBUF, b)


_sc_word = functools.partial(
    pl.kernel,
    out_type=jax.ShapeDtypeStruct((S, EMB_DIM), jnp.float32),
    mesh=_MESH,
    scratch_types=[
        pltpu.VMEM((TS,), jnp.int32),
        pltpu.VMEM((WNSUB, WSUB), jnp.int32),
        pltpu.VMEM((WNBUF, WSUB, EMB_DIM), jnp.float32),
        pltpu.SemaphoreType.DMA((WNBUF,)),
    ],
    compiler_params=pltpu.CompilerParams(needs_layout_passes=False,
                                         use_tc_tiling_on_sc=True),
)(_sc_word_body)


OSUB = 16
ONSUB = TS // OSUB
ONBUF = 3
_NIR = OSUB * G // 128


def _sc_oe_body(tok_hbm, oe_hbm, x_out, tok_v, idx_v, oe_rows, sems, osem):
    wid = lax.axis_index("s") * 2 + lax.axis_index("c")
    base = wid * TS

    @pl.when(wid == 0)
    def _():
        tok_v[pl.ds(0, 16)] = jnp.zeros((16,), jnp.int32)
        pltpu.sync_copy(tok_hbm.at[pl.ds(0, TS)], tok_v.at[pl.ds(8, TS)])

    @pl.when(wid != 0)
    def _():
        pltpu.sync_copy(tok_hbm.at[pl.ds(base - 8, TS + 8)],
                        tok_v.at[pl.ds(0, TS + 8)])

    lanes = lax.iota(jnp.int32, 16)

    def _hash_step(j, _):
        off0 = 8 + j * 16
        t0 = plsc.load_gather(tok_v, [lanes + off0]).astype(jnp.uint32)
        t1 = plsc.load_gather(tok_v, [lanes + (off0 - 1)]).astype(jnp.uint32)
        t2 = plsc.load_gather(tok_v, [lanes + (off0 - 2)]).astype(jnp.uint32)
        for g in range(G):
            mod = jnp.uint32(_MODS[g])
            w1 = jnp.uint32(pow(V_TEXT, 1, _MODS[g]))
            acc = t0 % mod
            acc = (acc + (t1 * w1) % mod) % mod
            if g >= 8:
                w2 = jnp.uint32(pow(V_TEXT, 2, _MODS[g]))
                acc = (acc + (t2 * w2) % mod) % mod
            ids = acc.astype(jnp.int32) + _OFFS[g]
            p = lanes * 16 + (j * 256 + g)
            plsc.store_scatter(idx_v, [p >> 7, p & 127], ids)
        return 0

    lax.fori_loop(0, TS // 16, _hash_step, 0)

    def _issue(sub, b):
        return [
            pltpu.async_copy(oe_hbm.at[idx_v.at[sub * _NIR + r]],
                             oe_rows.at[b, pl.ds(r * 128, 128)],
                             sems.at[b])
            for r in range(_NIR)
        ]

    copies = {}
    out_copies = {}
    for sub in range(min(ONBUF, ONSUB)):
        copies[sub] = _issue(sub, sub % ONBUF)
    for sub in range(ONSUB):
        b = sub % ONBUF
        for c in copies.pop(sub):
            c.wait()
        out_copies[sub] = pltpu.async_copy(
            oe_rows.at[b],
            x_out.at[pl.ds((base + sub * OSUB) * G, OSUB * G)],
            osem)
        nxt = sub + ONBUF
        if nxt < ONSUB:
            out_copies.pop(nxt - ONBUF).wait()
            copies[nxt] = _issue(nxt, nxt % ONBUF)
    for sub, c in out_copies.items():
        c.wait()


_sc_oe = functools.partial(
    pl.kernel,
    out_type=jax.ShapeDtypeStruct((S * G, H), jnp.float32),
    mesh=_MESH,
    scratch_types=[
        pltpu.VMEM((TS + 16,), jnp.int32),
        pltpu.VMEM((TS * G // 128, 128), jnp.int32),
        pltpu.VMEM((ONBUF, OSUB * G, H), jnp.float32),
        pltpu.SemaphoreType.DMA((ONBUF,)),
        pltpu.SemaphoreType.DMA,
    ],
    compiler_params=pltpu.CompilerParams(needs_layout_passes=False,
                                         use_tc_tiling_on_sc=False),
)(_sc_oe_body)


BS = 256


def _tc_body(tok_ref, x_ref, w_ref, wd_ref, o_ref):
    t = tok_ref[...]
    m = ((t != 0) & (t != 1) & (t != 2)).astype(jnp.float32)
    y = jnp.dot(x_ref[...], w_ref[...], preferred_element_type=jnp.float32)
    o_ref[...] = (wd_ref[...] + y * m) * (1.0 / 17.0)


def _tc_combine(tokens, x, wp, wd):
    return pl.pallas_call(
        _tc_body,
        grid=(S // BS,),
        in_specs=[
            pl.BlockSpec((BS, 1), lambda i: (i, 0)),
            pl.BlockSpec((BS, G * H), lambda i: (i, 0)),
            pl.BlockSpec((G * H, EMB_DIM), lambda i: (0, 0)),
            pl.BlockSpec((BS, EMB_DIM), lambda i: (i, 0)),
        ],
        out_specs=pl.BlockSpec((BS, EMB_DIM), lambda i: (i, 0)),
        out_shape=jax.ShapeDtypeStruct((S, EMB_DIM), jnp.float32),
    )(tokens.reshape(S, 1), x, wp, wd)


def kernel(input_ids, word_weight, oe_weight, oe_projection):
    tokens = input_ids.astype(jnp.int32)
    wd = _sc_word(tokens, word_weight)
    x = _sc_oe(tokens, oe_weight)
    wp = oe_projection.reshape(G * H, EMB_DIM)
    return _tc_combine(tokens, x.reshape(S, G * H), wp, wd)
